# Initial kernel scaffold; baseline (speedup 1.0000x reference)
import jax
import jax.numpy as jnp
from jax.experimental import pallas as pl


def kernel(X, W_loc):
    raise NotImplementedError("write your pallas kernel here")


# speedup vs baseline: 1.5212x; 1.5212x over previous
import functools

import jax
import jax.numpy as jnp
from jax import lax
from jax.experimental import pallas as pl
from jax.experimental.pallas import tpu as pltpu
from jax.experimental.pallas import tpu_sc as plsc

B = 2
H = W = D = 128
N = H * W * D
BN = B * N
KB = 4
NC, NS, L = 2, 16, 16
NW = NC * NS
PW = BN // NW
CH = 2048
NCHUNK = PW // CH


def _tc_body(a00, a01, a02, a10, a11, a12, w_ref, xl_ref, yl_ref, zl_ref,
             idx_ref, wgt_ref):
    b = pl.program_id(0)
    planes = ((a00, a01, a02), (a10, a11, a12))
    acc = [None, None, None]
    for ci in range(2):
        for dk in range(3):
            x = planes[ci][dk][...].astype(jnp.bfloat16).astype(jnp.float32)
            for dh in range(3):
                for dw in range(3):
                    sl = x[:, dh:dh + H, dw:dw + W]
                    for co in range(3):
                        w = w_ref[dk, dh, dw, ci, co]
                        t = sl * w.astype(jnp.bfloat16).astype(jnp.float32)
                        acc[co] = t if acc[co] is None else acc[co] + t
    def0, def1, def2 = acc

    xl = xl_ref[...][None, :, :]
    yl = yl_ref[...][None, :, :]
    zl = zl_ref[...][:, 0:1][:, :, None]
    x = 0.5 * ((xl + def0) + 1.0) * 128.0
    y = 0.5 * ((yl + def1) + 1.0) * 128.0
    z = 0.5 * ((zl + def2) + 1.0) * 128.0

    xi = jnp.floor(x).astype(jnp.int32)
    yi = jnp.floor(y).astype(jnp.int32)
    zi = jnp.floor(z).astype(jnp.int32)
    x0 = jnp.clip(xi, 0, W - 1)
    x1 = jnp.clip(xi + 1, 0, W - 1)
    y0 = jnp.clip(yi, 0, H - 1)
    y1 = jnp.clip(yi + 1, 0, H - 1)
    z0 = jnp.clip(zi, 0, D - 1)
    z1 = jnp.clip(zi + 1, 0, D - 1)

    base = b * N
    bz0 = base + z0 * (W * H)
    bz1 = base + z1 * (W * H)
    b00 = bz0 + y0 * W
    b01 = bz0 + y1 * W
    b10 = bz1 + y0 * W
    b11 = bz1 + y1 * W
    idx_ref[0] = b00 + x0
    idx_ref[1] = b00 + x1
    idx_ref[2] = b01 + x0
    idx_ref[3] = b01 + x1
    idx_ref[4] = b10 + x0
    idx_ref[5] = b10 + x1
    idx_ref[6] = b11 + x0
    idx_ref[7] = b11 + x1

    x0f = x0.astype(jnp.float32)
    x1f = x1.astype(jnp.float32)
    y0f = y0.astype(jnp.float32)
    y1f = y1.astype(jnp.float32)
    z0f = z0.astype(jnp.float32)
    z1f = z1.astype(jnp.float32)
    dxa = x1f - x
    dxb = x - x0f
    dya = y1f - y
    dyb = y - y0f
    dza = z1f - z
    dzb = z - z0f
    wa = dxa * dya
    wb = dxa * dyb
    wc = dxb * dya
    wd = dxb * dyb
    wgt_ref[0] = wa * dza
    wgt_ref[1] = wa * dzb
    wgt_ref[2] = wb * dza
    wgt_ref[3] = wb * dzb
    wgt_ref[4] = wc * dza
    wgt_ref[5] = wc * dzb
    wgt_ref[6] = wd * dza
    wgt_ref[7] = wd * dzb


def _tc_indices_weights(X, W_loc):
    Xt = jnp.pad(X.transpose(0, 3, 1, 2, 4),
                 ((0, 0), (1, 1), (1, 1), (1, 1), (0, 0)))
    ins = []
    for ci in range(2):
        for dk in range(3):
            ins.append(Xt[:, dk:dk + D, :, :, ci])
    Wt = W_loc.transpose(2, 0, 1, 3, 4)

    x_lin = jnp.linspace(-1.0, 1.0, W)
    y_lin = jnp.linspace(-1.0, 1.0, H)
    z_lin = jnp.linspace(-1.0, 1.0, D)
    XL = jnp.broadcast_to(x_lin[None, :], (H, W))
    YL = jnp.broadcast_to(y_lin[:, None], (H, W))
    ZL = jnp.broadcast_to(z_lin.reshape(D // KB, KB, 1), (D // KB, KB, W))

    a_spec = pl.BlockSpec((None, KB, H + 2, W + 2), lambda b, kc: (b, kc, 0, 0))
    out_spec = pl.BlockSpec((8, None, KB, H, W), lambda b, kc: (0, b, kc, 0, 0))
    idx, wgt = pl.pallas_call(
        _tc_body,
        grid=(B, D // KB),
        in_specs=[a_spec] * 6 + [
            pl.BlockSpec(memory_space=pltpu.SMEM),
            pl.BlockSpec((H, W), lambda b, kc: (0, 0)),
            pl.BlockSpec((H, W), lambda b, kc: (0, 0)),
            pl.BlockSpec((None, KB, W), lambda b, kc: (kc, 0, 0)),
        ],
        out_specs=[out_spec, out_spec],
        out_shape=[
            jax.ShapeDtypeStruct((8, B, D, H, W), jnp.int32),
            jax.ShapeDtypeStruct((8, B, D, H, W), jnp.float32),
        ],
    )(*ins, Wt, XL, YL, ZL)
    return idx.reshape(8, BN), wgt.reshape(8, BN)


def _sc_body(idx_hbm, wgt_hbm, tab_hbm, out_hbm, *scratch):
    idx_v = scratch[0:8]
    val_v = scratch[8:16]
    wgt_v, out_v, sem = scratch[16:19]
    wid = lax.axis_index("s") * NC + lax.axis_index("c")

    def chunk_body(ci, carry):
        start = wid * PW + ci * CH
        for t in range(8):
            pltpu.sync_copy(idx_hbm.at[t, pl.ds(start, CH)], idx_v[t])
        pltpu.sync_copy(wgt_hbm.at[:, pl.ds(start, CH)], wgt_v)
        def gather_body(gi, c1):
            o = pl.ds(gi * 128, 128)
            copies = [
                pltpu.async_copy(tab_hbm.at[idx_v[t].at[o]], val_v[t].at[o], sem)
                for t in range(8)
            ]
            for c in copies:
                c.wait()
            return c1

        lax.fori_loop(0, CH // 128, gather_body, 0)

        def vec_body(vi, c2):
            sl = pl.ds(vi * L, L)
            s = wgt_v[0, sl] * val_v[0][sl]
            for t in range(1, 8):
                s = s + wgt_v[t, sl] * val_v[t][sl]
            out_v[sl] = s
            return c2

        lax.fori_loop(0, CH // L, vec_body, 0)
        pltpu.sync_copy(out_v, out_hbm.at[pl.ds(start, CH)])
        return carry

    lax.fori_loop(0, NCHUNK, chunk_body, 0)


@functools.cache
def _sc_interp():
    return pl.kernel(
        _sc_body,
        mesh=plsc.VectorSubcoreMesh(core_axis_name="c", subcore_axis_name="s"),
        out_type=jax.ShapeDtypeStruct((BN,), jnp.float32),
        scratch_types=(
            [pltpu.VMEM((CH,), jnp.int32)] * 8
            + [pltpu.VMEM((CH,), jnp.float32)] * 8
            + [
                pltpu.VMEM((8, CH), jnp.float32),
                pltpu.VMEM((CH,), jnp.float32),
                pltpu.SemaphoreType.DMA,
            ]
        ),
    )


def kernel(X, W_loc):
    idx, wgt = _tc_indices_weights(X, W_loc)
    tab = X[..., 0].reshape(BN)
    out_t = _sc_interp()(idx, wgt, tab)
    return out_t.reshape(B, D, H, W).transpose(0, 2, 3, 1)[..., None]


# speedup vs baseline: 1.7496x; 1.1502x over previous
import functools

import jax
import jax.numpy as jnp
from jax import lax
from jax.experimental import pallas as pl
from jax.experimental.pallas import tpu as pltpu
from jax.experimental.pallas import tpu_sc as plsc

B = 2
H = W = D = 128
N = H * W * D
BN = B * N
KB = 4
NC, NS, L = 2, 16, 16
NW = NC * NS
PW = BN // NW
CH = 2048
NCHUNK = PW // CH


def _tc_body(a00, a01, a02, a10, a11, a12, w_ref, xl_ref, yl_ref, zl_ref,
             idx_ref, wgt_ref):
    b = pl.program_id(0)
    planes = ((a00, a01, a02), (a10, a11, a12))
    acc = [None, None, None]
    for ci in range(2):
        for dk in range(3):
            x = planes[ci][dk][...].astype(jnp.bfloat16).astype(jnp.float32)
            for dh in range(3):
                for dw in range(3):
                    sl = x[:, dh:dh + H, dw:dw + W]
                    for co in range(3):
                        w = w_ref[dk, dh, dw, ci, co]
                        t = sl * w.astype(jnp.bfloat16).astype(jnp.float32)
                        acc[co] = t if acc[co] is None else acc[co] + t
    def0, def1, def2 = acc

    xl = xl_ref[...][None, :, :]
    yl = yl_ref[...][None, :, :]
    zl = zl_ref[...][:, 0:1][:, :, None]
    x = 0.5 * ((xl + def0) + 1.0) * 128.0
    y = 0.5 * ((yl + def1) + 1.0) * 128.0
    z = 0.5 * ((zl + def2) + 1.0) * 128.0

    xi = jnp.floor(x).astype(jnp.int32)
    yi = jnp.floor(y).astype(jnp.int32)
    zi = jnp.floor(z).astype(jnp.int32)
    x0 = jnp.clip(xi, 0, W - 1)
    x1 = jnp.clip(xi + 1, 0, W - 1)
    y0 = jnp.clip(yi, 0, H - 1)
    y1 = jnp.clip(yi + 1, 0, H - 1)
    z0 = jnp.clip(zi, 0, D - 1)
    z1 = jnp.clip(zi + 1, 0, D - 1)

    base = b * N
    bz0 = base + z0 * (W * H)
    bz1 = base + z1 * (W * H)
    b00 = bz0 + y0 * W
    b01 = bz0 + y1 * W
    b10 = bz1 + y0 * W
    b11 = bz1 + y1 * W
    idx_ref[0] = b00 + x0
    idx_ref[1] = b00 + x1
    idx_ref[2] = b01 + x0
    idx_ref[3] = b01 + x1
    idx_ref[4] = b10 + x0
    idx_ref[5] = b10 + x1
    idx_ref[6] = b11 + x0
    idx_ref[7] = b11 + x1

    x0f = x0.astype(jnp.float32)
    x1f = x1.astype(jnp.float32)
    y0f = y0.astype(jnp.float32)
    y1f = y1.astype(jnp.float32)
    z0f = z0.astype(jnp.float32)
    z1f = z1.astype(jnp.float32)
    dxa = x1f - x
    dxb = x - x0f
    dya = y1f - y
    dyb = y - y0f
    dza = z1f - z
    dzb = z - z0f
    wa = dxa * dya
    wb = dxa * dyb
    wc = dxb * dya
    wd = dxb * dyb
    wgt_ref[0] = wa * dza
    wgt_ref[1] = wa * dzb
    wgt_ref[2] = wb * dza
    wgt_ref[3] = wb * dzb
    wgt_ref[4] = wc * dza
    wgt_ref[5] = wc * dzb
    wgt_ref[6] = wd * dza
    wgt_ref[7] = wd * dzb


def _tc_indices_weights(X, W_loc):
    Xt = jnp.pad(X.transpose(0, 3, 1, 2, 4),
                 ((0, 0), (1, 1), (1, 1), (1, 1), (0, 0)))
    ins = []
    for ci in range(2):
        for dk in range(3):
            ins.append(Xt[:, dk:dk + D, :, :, ci])
    Wt = W_loc.transpose(2, 0, 1, 3, 4)

    x_lin = jnp.linspace(-1.0, 1.0, W)
    y_lin = jnp.linspace(-1.0, 1.0, H)
    z_lin = jnp.linspace(-1.0, 1.0, D)
    XL = jnp.broadcast_to(x_lin[None, :], (H, W))
    YL = jnp.broadcast_to(y_lin[:, None], (H, W))
    ZL = jnp.broadcast_to(z_lin.reshape(D // KB, KB, 1), (D // KB, KB, W))

    a_spec = pl.BlockSpec((None, KB, H + 2, W + 2), lambda b, kc: (b, kc, 0, 0))
    out_spec = pl.BlockSpec((8, None, KB, H, W), lambda b, kc: (0, b, kc, 0, 0))
    idx, wgt = pl.pallas_call(
        _tc_body,
        grid=(B, D // KB),
        in_specs=[a_spec] * 6 + [
            pl.BlockSpec(memory_space=pltpu.SMEM),
            pl.BlockSpec((H, W), lambda b, kc: (0, 0)),
            pl.BlockSpec((H, W), lambda b, kc: (0, 0)),
            pl.BlockSpec((None, KB, W), lambda b, kc: (kc, 0, 0)),
        ],
        out_specs=[out_spec, out_spec],
        out_shape=[
            jax.ShapeDtypeStruct((8, B, D, H, W), jnp.int32),
            jax.ShapeDtypeStruct((8, B, D, H, W), jnp.float32),
        ],
    )(*ins, Wt, XL, YL, ZL)
    return idx.reshape(8, BN), wgt.reshape(8, BN)


def _sc_body(idx_hbm, wgt_hbm, tab_hbm, out_hbm, *scratch):
    idx_v = scratch[0:8]
    val_v = scratch[8:16]
    wgt_v, out_v, sem = scratch[16:19]
    wid = lax.axis_index("s") * NC + lax.axis_index("c")

    def chunk_body(ci, carry):
        start = wid * PW + ci * CH
        for t in range(8):
            pltpu.sync_copy(idx_hbm.at[t, pl.ds(start, CH)], idx_v[t])
        pltpu.sync_copy(wgt_hbm.at[:, pl.ds(start, CH)], wgt_v)
        copies = []
        for gi in range(CH // 128):
            o = pl.ds(gi * 128, 128)
            for t in range(8):
                copies.append(
                    pltpu.async_copy(tab_hbm.at[idx_v[t].at[o]], val_v[t].at[o], sem)
                )
        for c in copies:
            c.wait()

        def vec_body(vi, c2):
            sl = pl.ds(vi * L, L)
            s = wgt_v[0, sl] * val_v[0][sl]
            for t in range(1, 8):
                s = s + wgt_v[t, sl] * val_v[t][sl]
            out_v[sl] = s
            return c2

        lax.fori_loop(0, CH // L, vec_body, 0)
        pltpu.sync_copy(out_v, out_hbm.at[pl.ds(start, CH)])
        return carry

    lax.fori_loop(0, NCHUNK, chunk_body, 0)


@functools.cache
def _sc_interp():
    return pl.kernel(
        _sc_body,
        mesh=plsc.VectorSubcoreMesh(core_axis_name="c", subcore_axis_name="s"),
        out_type=jax.ShapeDtypeStruct((BN,), jnp.float32),
        scratch_types=(
            [pltpu.VMEM((CH,), jnp.int32)] * 8
            + [pltpu.VMEM((CH,), jnp.float32)] * 8
            + [
                pltpu.VMEM((8, CH), jnp.float32),
                pltpu.VMEM((CH,), jnp.float32),
                pltpu.SemaphoreType.DMA,
            ]
        ),
    )


def kernel(X, W_loc):
    idx, wgt = _tc_indices_weights(X, W_loc)
    tab = X[..., 0].reshape(BN)
    out_t = _sc_interp()(idx, wgt, tab)
    return out_t.reshape(B, D, H, W).transpose(0, 2, 3, 1)[..., None]


# speedup vs baseline: 1.7872x; 1.0215x over previous
import functools

import jax
import jax.numpy as jnp
from jax import lax
from jax.experimental import pallas as pl
from jax.experimental.pallas import tpu as pltpu
from jax.experimental.pallas import tpu_sc as plsc

B = 2
H = W = D = 128
N = H * W * D
BN = B * N
KB = 4
NC, NS, L = 2, 16, 16
NW = NC * NS
PW = BN // NW
CH = 4096
NCHUNK = PW // CH


def _tc_body(a00, a01, a02, a10, a11, a12, w_ref, xl_ref, yl_ref, zl_ref,
             idx_ref, wgt_ref):
    b = pl.program_id(0)
    planes = ((a00, a01, a02), (a10, a11, a12))
    acc = [None, None, None]
    for ci in range(2):
        for dk in range(3):
            x = planes[ci][dk][...].astype(jnp.bfloat16).astype(jnp.float32)
            for dh in range(3):
                for dw in range(3):
                    sl = x[:, dh:dh + H, dw:dw + W]
                    for co in range(3):
                        w = w_ref[dk, dh, dw, ci, co]
                        t = sl * w.astype(jnp.bfloat16).astype(jnp.float32)
                        acc[co] = t if acc[co] is None else acc[co] + t
    def0, def1, def2 = acc

    xl = xl_ref[...][None, :, :]
    yl = yl_ref[...][None, :, :]
    zl = zl_ref[...][:, 0:1][:, :, None]
    x = 0.5 * ((xl + def0) + 1.0) * 128.0
    y = 0.5 * ((yl + def1) + 1.0) * 128.0
    z = 0.5 * ((zl + def2) + 1.0) * 128.0

    xi = jnp.floor(x).astype(jnp.int32)
    yi = jnp.floor(y).astype(jnp.int32)
    zi = jnp.floor(z).astype(jnp.int32)
    x0 = jnp.clip(xi, 0, W - 1)
    x1 = jnp.clip(xi + 1, 0, W - 1)
    y0 = jnp.clip(yi, 0, H - 1)
    y1 = jnp.clip(yi + 1, 0, H - 1)
    z0 = jnp.clip(zi, 0, D - 1)
    z1 = jnp.clip(zi + 1, 0, D - 1)

    base = b * N
    bz0 = base + z0 * (W * H)
    bz1 = base + z1 * (W * H)
    b00 = bz0 + y0 * W
    b01 = bz0 + y1 * W
    b10 = bz1 + y0 * W
    b11 = bz1 + y1 * W
    idx_ref[0] = b00 + x0
    idx_ref[1] = b00 + x1
    idx_ref[2] = b01 + x0
    idx_ref[3] = b01 + x1
    idx_ref[4] = b10 + x0
    idx_ref[5] = b10 + x1
    idx_ref[6] = b11 + x0
    idx_ref[7] = b11 + x1

    x0f = x0.astype(jnp.float32)
    x1f = x1.astype(jnp.float32)
    y0f = y0.astype(jnp.float32)
    y1f = y1.astype(jnp.float32)
    z0f = z0.astype(jnp.float32)
    z1f = z1.astype(jnp.float32)
    dxa = x1f - x
    dxb = x - x0f
    dya = y1f - y
    dyb = y - y0f
    dza = z1f - z
    dzb = z - z0f
    wa = dxa * dya
    wb = dxa * dyb
    wc = dxb * dya
    wd = dxb * dyb
    wgt_ref[0] = wa * dza
    wgt_ref[1] = wa * dzb
    wgt_ref[2] = wb * dza
    wgt_ref[3] = wb * dzb
    wgt_ref[4] = wc * dza
    wgt_ref[5] = wc * dzb
    wgt_ref[6] = wd * dza
    wgt_ref[7] = wd * dzb


def _tc_indices_weights(X, W_loc):
    Xt = jnp.pad(X.transpose(0, 3, 1, 2, 4),
                 ((0, 0), (1, 1), (1, 1), (1, 1), (0, 0)))
    ins = []
    for ci in range(2):
        for dk in range(3):
            ins.append(Xt[:, dk:dk + D, :, :, ci])
    Wt = W_loc.transpose(2, 0, 1, 3, 4)

    x_lin = jnp.linspace(-1.0, 1.0, W)
    y_lin = jnp.linspace(-1.0, 1.0, H)
    z_lin = jnp.linspace(-1.0, 1.0, D)
    XL = jnp.broadcast_to(x_lin[None, :], (H, W))
    YL = jnp.broadcast_to(y_lin[:, None], (H, W))
    ZL = jnp.broadcast_to(z_lin.reshape(D // KB, KB, 1), (D // KB, KB, W))

    a_spec = pl.BlockSpec((None, KB, H + 2, W + 2), lambda b, kc: (b, kc, 0, 0))
    out_spec = pl.BlockSpec((8, None, KB, H, W), lambda b, kc: (0, b, kc, 0, 0))
    idx, wgt = pl.pallas_call(
        _tc_body,
        grid=(B, D // KB),
        in_specs=[a_spec] * 6 + [
            pl.BlockSpec(memory_space=pltpu.SMEM),
            pl.BlockSpec((H, W), lambda b, kc: (0, 0)),
            pl.BlockSpec((H, W), lambda b, kc: (0, 0)),
            pl.BlockSpec((None, KB, W), lambda b, kc: (kc, 0, 0)),
        ],
        out_specs=[out_spec, out_spec],
        out_shape=[
            jax.ShapeDtypeStruct((8, B, D, H, W), jnp.int32),
            jax.ShapeDtypeStruct((8, B, D, H, W), jnp.float32),
        ],
    )(*ins, Wt, XL, YL, ZL)
    return idx.reshape(8, BN), wgt.reshape(8, BN)


def _sc_body(idx_hbm, wgt_hbm, tab_hbm, out_hbm, *scratch):
    idx_v = scratch[0:8]
    val_v = scratch[8:16]
    wgt_v, out_v, sem = scratch[16:19]
    wid = lax.axis_index("s") * NC + lax.axis_index("c")

    def chunk_body(ci, carry):
        start = wid * PW + ci * CH
        for t in range(8):
            pltpu.sync_copy(idx_hbm.at[t, pl.ds(start, CH)], idx_v[t])
        pltpu.sync_copy(wgt_hbm.at[:, pl.ds(start, CH)], wgt_v)
        copies = []
        for gi in range(CH // 128):
            o = pl.ds(gi * 128, 128)
            for t in range(8):
                copies.append(
                    pltpu.async_copy(tab_hbm.at[idx_v[t].at[o]], val_v[t].at[o], sem)
                )
        for c in copies:
            c.wait()

        def vec_body(vi, c2):
            sl = pl.ds(vi * L, L)
            s = wgt_v[0, sl] * val_v[0][sl]
            for t in range(1, 8):
                s = s + wgt_v[t, sl] * val_v[t][sl]
            out_v[sl] = s
            return c2

        lax.fori_loop(0, CH // L, vec_body, 0)
        pltpu.sync_copy(out_v, out_hbm.at[pl.ds(start, CH)])
        return carry

    lax.fori_loop(0, NCHUNK, chunk_body, 0)


@functools.cache
def _sc_interp():
    return pl.kernel(
        _sc_body,
        mesh=plsc.VectorSubcoreMesh(core_axis_name="c", subcore_axis_name="s"),
        out_type=jax.ShapeDtypeStruct((BN,), jnp.float32),
        scratch_types=(
            [pltpu.VMEM((CH,), jnp.int32)] * 8
            + [pltpu.VMEM((CH,), jnp.float32)] * 8
            + [
                pltpu.VMEM((8, CH), jnp.float32),
                pltpu.VMEM((CH,), jnp.float32),
                pltpu.SemaphoreType.DMA,
            ]
        ),
    )


def kernel(X, W_loc):
    idx, wgt = _tc_indices_weights(X, W_loc)
    tab = X[..., 0].reshape(BN)
    out_t = _sc_interp()(idx, wgt, tab)
    return out_t.reshape(B, D, H, W).transpose(0, 2, 3, 1)[..., None]


# speedup vs baseline: 1.8237x; 1.0204x over previous
import functools

import jax
import jax.numpy as jnp
from jax import lax
from jax.experimental import pallas as pl
from jax.experimental.pallas import tpu as pltpu
from jax.experimental.pallas import tpu_sc as plsc

B = 2
H = W = D = 128
N = H * W * D
BN = B * N
KB = 4
NC, NS, L = 2, 16, 16
NW = NC * NS
PW = BN // NW
CH = 4096
NCHUNK = PW // CH


def _tc_body(a00, a01, a02, a10, a11, a12, w_ref, xl_ref, yl_ref, zl_ref,
             idx_ref, wgt_ref):
    b = pl.program_id(0)
    planes = ((a00, a01, a02), (a10, a11, a12))
    acc = [None, None, None]
    for ci in range(2):
        for dk in range(3):
            x = planes[ci][dk][...].astype(jnp.bfloat16).astype(jnp.float32)
            for dh in range(3):
                for dw in range(3):
                    sl = x[:, dh:dh + H, dw:dw + W]
                    for co in range(3):
                        w = w_ref[dk, dh, dw, ci, co]
                        t = sl * w.astype(jnp.bfloat16).astype(jnp.float32)
                        acc[co] = t if acc[co] is None else acc[co] + t
    def0, def1, def2 = acc

    xl = xl_ref[...][None, :, :]
    yl = yl_ref[...][None, :, :]
    zl = zl_ref[...][:, 0:1][:, :, None]
    x = 0.5 * ((xl + def0) + 1.0) * 128.0
    y = 0.5 * ((yl + def1) + 1.0) * 128.0
    z = 0.5 * ((zl + def2) + 1.0) * 128.0

    xi = jnp.floor(x).astype(jnp.int32)
    yi = jnp.floor(y).astype(jnp.int32)
    zi = jnp.floor(z).astype(jnp.int32)
    x0 = jnp.clip(xi, 0, W - 1)
    x1 = jnp.clip(xi + 1, 0, W - 1)
    y0 = jnp.clip(yi, 0, H - 1)
    y1 = jnp.clip(yi + 1, 0, H - 1)
    z0 = jnp.clip(zi, 0, D - 1)
    z1 = jnp.clip(zi + 1, 0, D - 1)

    base = b * N
    bz0 = base + z0 * (W * H)
    bz1 = base + z1 * (W * H)
    b00 = bz0 + y0 * W
    b01 = bz0 + y1 * W
    b10 = bz1 + y0 * W
    b11 = bz1 + y1 * W
    idx_ref[0] = b00 + x0
    idx_ref[1] = b00 + x1
    idx_ref[2] = b01 + x0
    idx_ref[3] = b01 + x1
    idx_ref[4] = b10 + x0
    idx_ref[5] = b10 + x1
    idx_ref[6] = b11 + x0
    idx_ref[7] = b11 + x1

    x0f = x0.astype(jnp.float32)
    x1f = x1.astype(jnp.float32)
    y0f = y0.astype(jnp.float32)
    y1f = y1.astype(jnp.float32)
    z0f = z0.astype(jnp.float32)
    z1f = z1.astype(jnp.float32)
    dxa = x1f - x
    dxb = x - x0f
    dya = y1f - y
    dyb = y - y0f
    dza = z1f - z
    dzb = z - z0f
    wa = dxa * dya
    wb = dxa * dyb
    wc = dxb * dya
    wd = dxb * dyb
    wgt_ref[0] = wa * dza
    wgt_ref[1] = wa * dzb
    wgt_ref[2] = wb * dza
    wgt_ref[3] = wb * dzb
    wgt_ref[4] = wc * dza
    wgt_ref[5] = wc * dzb
    wgt_ref[6] = wd * dza
    wgt_ref[7] = wd * dzb


def _tc_indices_weights(X, W_loc):
    Xt = jnp.pad(X.transpose(0, 3, 1, 2, 4),
                 ((0, 0), (1, 1), (1, 1), (1, 1), (0, 0)))
    ins = []
    for ci in range(2):
        for dk in range(3):
            ins.append(Xt[:, dk:dk + D, :, :, ci])
    Wt = W_loc.transpose(2, 0, 1, 3, 4)

    x_lin = jnp.linspace(-1.0, 1.0, W)
    y_lin = jnp.linspace(-1.0, 1.0, H)
    z_lin = jnp.linspace(-1.0, 1.0, D)
    XL = jnp.broadcast_to(x_lin[None, :], (H, W))
    YL = jnp.broadcast_to(y_lin[:, None], (H, W))
    ZL = jnp.broadcast_to(z_lin.reshape(D // KB, KB, 1), (D // KB, KB, W))

    a_spec = pl.BlockSpec((None, KB, H + 2, W + 2), lambda b, kc: (b, kc, 0, 0))
    out_spec = pl.BlockSpec((8, None, KB, H, W), lambda b, kc: (0, b, kc, 0, 0))
    idx, wgt = pl.pallas_call(
        _tc_body,
        grid=(B, D // KB),
        in_specs=[a_spec] * 6 + [
            pl.BlockSpec(memory_space=pltpu.SMEM),
            pl.BlockSpec((H, W), lambda b, kc: (0, 0)),
            pl.BlockSpec((H, W), lambda b, kc: (0, 0)),
            pl.BlockSpec((None, KB, W), lambda b, kc: (kc, 0, 0)),
        ],
        out_specs=[out_spec, out_spec],
        out_shape=[
            jax.ShapeDtypeStruct((8, B, D, H, W), jnp.int32),
            jax.ShapeDtypeStruct((8, B, D, H, W), jnp.float32),
        ],
    )(*ins, Wt, XL, YL, ZL)
    return idx.reshape(8, BN), wgt.reshape(8, BN)


def _sc_body(idx_hbm, wgt_hbm, tab_hbm, out_hbm, *scratch):
    idx_v = scratch[0:8]
    val_v = scratch[8:16]
    wgt_v, out_v, sem = scratch[16:19]
    wid = lax.axis_index("s") * NC + lax.axis_index("c")

    def chunk_body(ci, carry):
        start = wid * PW + ci * CH
        stage = [
            pltpu.async_copy(idx_hbm.at[t, pl.ds(start, CH)], idx_v[t], sem)
            for t in range(8)
        ]
        stage.append(pltpu.async_copy(wgt_hbm.at[:, pl.ds(start, CH)], wgt_v, sem))
        for c in stage:
            c.wait()
        copies = []
        for gi in range(CH // 128):
            o = pl.ds(gi * 128, 128)
            for t in range(8):
                copies.append(
                    pltpu.async_copy(tab_hbm.at[idx_v[t].at[o]], val_v[t].at[o], sem)
                )
        for c in copies:
            c.wait()

        def vec_body(vi, c2):
            sl = pl.ds(vi * L, L)
            s = wgt_v[0, sl] * val_v[0][sl]
            for t in range(1, 8):
                s = s + wgt_v[t, sl] * val_v[t][sl]
            out_v[sl] = s
            return c2

        lax.fori_loop(0, CH // L, vec_body, 0)
        pltpu.sync_copy(out_v, out_hbm.at[pl.ds(start, CH)])
        return carry

    lax.fori_loop(0, NCHUNK, chunk_body, 0)


@functools.cache
def _sc_interp():
    return pl.kernel(
        _sc_body,
        mesh=plsc.VectorSubcoreMesh(core_axis_name="c", subcore_axis_name="s"),
        out_type=jax.ShapeDtypeStruct((BN,), jnp.float32),
        scratch_types=(
            [pltpu.VMEM((CH,), jnp.int32)] * 8
            + [pltpu.VMEM((CH,), jnp.float32)] * 8
            + [
                pltpu.VMEM((8, CH), jnp.float32),
                pltpu.VMEM((CH,), jnp.float32),
                pltpu.SemaphoreType.DMA,
            ]
        ),
    )


def kernel(X, W_loc):
    idx, wgt = _tc_indices_weights(X, W_loc)
    tab = X[..., 0].reshape(BN)
    out_t = _sc_interp()(idx, wgt, tab)
    return out_t.reshape(B, D, H, W).transpose(0, 2, 3, 1)[..., None]
